# Initial kernel scaffold; baseline (speedup 1.0000x reference)
import jax
import jax.numpy as jnp
from jax.experimental import pallas as pl


def kernel(xyz, new_xyz, features):
    raise NotImplementedError("write your pallas kernel here")


# speedup vs baseline: 10.5573x; 10.5573x over previous
import functools

import jax
import jax.numpy as jnp
import numpy as np
from jax import lax
from jax.experimental import pallas as pl
from jax.experimental.pallas import tpu as pltpu
from jax.experimental.pallas import tpu_sc as plsc

B, N, NP, C = 4, 8192, 2048, 64
S = 32
CO = C + 3
L = 16
NCHUNK = 32
TPB = 8
CPT = NP // TPB
PCH = 512
TASKS = 2 * CO
RADIUS2 = np.float32(0.2 * 0.2)


def _qag_body(xyz_f, cen_f, feats_f, out,
              xyz_v, cen_v, buf_v, idxc_v, row_v, cenrow_v, idxb_v, ob_v,
              idx_sh):
    ci = lax.axis_index("c")
    sid = lax.axis_index("s")
    zeros16 = jnp.zeros((L,), jnp.int32)
    iota16 = lax.iota(jnp.int32, L)

    lb = sid // TPB
    b = 2 * ci + lb
    c0 = (sid % TPB) * CPT
    pltpu.sync_copy(xyz_f.at[b], xyz_v)
    pltpu.sync_copy(cen_f.at[b], cen_v)

    def per_center(i, carry):
        col = jnp.full((L,), c0 + i, jnp.int32)
        cxv = plsc.load_gather(cen_v, [col])
        cyv = plsc.load_gather(cen_v, [col + NP])
        czv = plsc.load_gather(cen_v, [col + 2 * NP])
        buf_v[pl.ds(0, L)] = zeros16

        def cond(st):
            j, cnt = st
            return jnp.logical_and(j < N // NCHUNK, cnt < S)

        def body(st):
            j, cnt = st
            base = j * NCHUNK
            for u in range(NCHUNK // L):
                off = base + u * L
                xv = xyz_v[pl.ds(off, L)]
                yv = xyz_v[pl.ds(N + off, L)]
                zv = xyz_v[pl.ds(2 * N + off, L)]
                dx = xv - cxv
                dy = yv - cyv
                dz = zv - czv
                d2 = dx * dx + dy * dy + dz * dz
                m = d2 < RADIUS2
                mi = m.astype(jnp.int32)
                slot = cnt + plsc.cumsum(mi) - 1
                plsc.store_scatter(buf_v, [slot], iota16 + off, mask=m)
                cnt = cnt + jnp.sum(mi)
            return (j + 1, cnt)

        _, cnt = lax.while_loop(cond, body, (jnp.int32(0), jnp.int32(0)))
        firstv = plsc.load_gather(buf_v, [zeros16])
        for h in range(S // L):
            p = iota16 + h * L
            got = plsc.load_gather(buf_v, [p])
            idxc_v[pl.ds(i * S + h * L, L)] = jnp.where(p < cnt, got, firstv)
        return carry

    lax.fori_loop(0, CPT, per_center, 0)
    pltpu.sync_copy(idxc_v, idx_sh.at[lb, pl.ds(c0 * S, CPT * S)])
    plsc.subcore_barrier()

    n_rounds = -(-TASKS // 16)
    for jt in range(n_rounds):
        t = sid + jt * 16

        @pl.when(t < TASKS)
        def _run_task(t=t):
            lb2 = t // CO
            b2 = 2 * ci + lb2
            c = t % CO
            is_xyz = c < 3

            @pl.when(is_xyz)
            def _():
                pltpu.sync_copy(xyz_f.at[b2, pl.ds(c * N, N)], row_v)
                pltpu.sync_copy(cen_f.at[b2, pl.ds(c * NP, NP)], cenrow_v)

            @pl.when(jnp.logical_not(is_xyz))
            def _():
                pltpu.sync_copy(feats_f.at[b2, pl.ds((c - 3) * N, N)], row_v)

            for p0 in range(0, NP, PCH):
                pltpu.sync_copy(idx_sh.at[lb2, pl.ds(p0 * S, PCH * S)],
                                idxb_v)

                def per_np(q, carry):
                    cv = plsc.load_gather(
                        cenrow_v, [jnp.full((L,), p0 + q, jnp.int32)])
                    for h in range(S // L):
                        iv = idxb_v[pl.ds(q * S + h * L, L)]
                        vals = plsc.load_gather(row_v, [iv])
                        vals = jnp.where(is_xyz, vals - cv, vals)
                        ob_v[pl.ds(q * S + h * L, L)] = vals
                    return carry

                lax.fori_loop(0, PCH, per_np, 0)
                pltpu.sync_copy(
                    ob_v, out.at[b2, pl.ds((c * NP + p0) * S, PCH * S)])


@functools.cache
def _qag():
    return pl.kernel(
        _qag_body,
        out_type=jax.ShapeDtypeStruct((B, CO * NP * S), jnp.float32),
        mesh=plsc.VectorSubcoreMesh(core_axis_name="c", subcore_axis_name="s",
                                    num_cores=2, num_subcores=16),
        compiler_params=pltpu.CompilerParams(needs_layout_passes=False),
        scratch_types=[
            pltpu.VMEM((3 * N,), jnp.float32),
            pltpu.VMEM((3 * NP,), jnp.float32),
            pltpu.VMEM((64,), jnp.int32),
            pltpu.VMEM((CPT * S,), jnp.int32),
            pltpu.VMEM((N,), jnp.float32),
            pltpu.VMEM((NP,), jnp.float32),
            pltpu.VMEM((PCH * S,), jnp.int32),
            pltpu.VMEM((PCH * S,), jnp.float32),
            pltpu.VMEM_SHARED((2, NP * S), jnp.int32),
        ],
    )


def kernel(xyz, new_xyz, features):
    xyz_f = jnp.transpose(xyz, (0, 2, 1)).reshape(B, 3 * N)
    cen_f = jnp.transpose(new_xyz, (0, 2, 1)).reshape(B, 3 * NP)
    feats_f = features.reshape(B, C * N)
    out = _qag()(xyz_f, cen_f, feats_f)
    return out.reshape(B, CO, NP, S)


# speedup vs baseline: 34.3329x; 3.2521x over previous
import functools

import jax
import jax.numpy as jnp
import numpy as np
from jax import lax
from jax.experimental import pallas as pl
from jax.experimental.pallas import tpu as pltpu
from jax.experimental.pallas import tpu_sc as plsc

B, N, NP, C = 4, 8192, 2048, 64
S = 32
CO = C + 3
L = 16
TPB = 8
CPT = NP // TPB
GPM = 8
MACROS = N // (GPM * L)
BUFSZ = 160
FLUSH = 64
PCH = 256
RADIUS2 = np.float32(0.2 * 0.2)


def _qag_body(xyz_f, cen_f, feats_f, out,
              xyz_v, cen_v, buf_v, idxc_v, row_v, cenrow_v, idxi_v, ob_v,
              semi0, semi1, semo0, semo1, idx_sh):
    ci = lax.axis_index("c")
    sid = lax.axis_index("s")
    zeros16 = jnp.zeros((L,), jnp.int32)
    iota16 = lax.iota(jnp.int32, L)

    lb = sid // TPB
    b = 2 * ci + lb
    c0 = (sid % TPB) * CPT
    pltpu.sync_copy(xyz_f.at[pl.ds(b * 3 * N, 3 * N)], xyz_v)
    pltpu.sync_copy(cen_f.at[pl.ds(b * 3 * NP, 3 * NP)], cen_v)

    def pair_body(blk, ip, carry0):
        gA = blk * FLUSH + 2 * ip
        colA = jnp.full((L,), c0 + gA, jnp.int32)
        colB = colA + 1
        cxa = plsc.load_gather(cen_v, [colA])
        cya = plsc.load_gather(cen_v, [colA + NP])
        cza = plsc.load_gather(cen_v, [colA + 2 * NP])
        cxb = plsc.load_gather(cen_v, [colB])
        cyb = plsc.load_gather(cen_v, [colB + NP])
        czb = plsc.load_gather(cen_v, [colB + 2 * NP])
        buf_v[pl.ds(0, L)] = zeros16
        buf_v[pl.ds(BUFSZ, L)] = zeros16

        def cond(st):
            j, need, _, _ = st
            return jnp.logical_and(j < MACROS, need)

        def body(st):
            j, _, cntA, cntB = st
            base = j * (GPM * L)
            for k in range(GPM):
                off = base + k * L
                xv = xyz_v[pl.ds(off, L)]
                yv = xyz_v[pl.ds(N + off, L)]
                zv = xyz_v[pl.ds(2 * N + off, L)]
                iv = iota16 + off
                for cx, cy, cz, bb, which in (
                        (cxa, cya, cza, 0, 0), (cxb, cyb, czb, BUFSZ, 1)):
                    dx = xv - cx
                    dy = yv - cy
                    dz = zv - cz
                    d2 = dx * dx + dy * dy + dz * dz
                    m = d2 < RADIUS2
                    mi = m.astype(jnp.int32)
                    cnt = cntB if which else cntA
                    slot = jnp.minimum(cnt + plsc.cumsum(mi) - 1, BUFSZ - 1)
                    plsc.store_scatter(buf_v, [slot + bb], iv, mask=m)
                    cnt = cnt + plsc.all_reduce_population_count(m)
                    if which:
                        cntB = cnt
                    else:
                        cntA = cnt
            mn = jnp.minimum(cntA, cntB)
            need = mn[0] < S
            return (j + 1, need, cntA, cntB)

        init = (jnp.int32(0), jnp.bool_(True), zeros16, zeros16)
        _j, _need, cntA, cntB = lax.while_loop(cond, body, init)

        for bb, cnt, row in ((0, cntA, 2 * ip), (BUFSZ, cntB, 2 * ip + 1)):
            firstv = plsc.load_gather(buf_v, [zeros16 + bb])
            for h in range(S // L):
                p = iota16 + h * L
                got = plsc.load_gather(buf_v, [p + bb])
                idxc_v[pl.ds(row * S + h * L, L)] = jnp.where(
                    p < cnt, got, firstv)
        return carry0

    for blk in range(CPT // FLUSH):
        lax.fori_loop(0, FLUSH // 2,
                      functools.partial(pair_body, blk), 0)
        pltpu.sync_copy(
            idxc_v, idx_sh.at[lb, pl.ds((c0 + blk * FLUSH) * S, FLUSH * S)])
    plsc.subcore_barrier()

    semi = (semi0, semi1)
    semo = (semo0, semo1)
    n_chunks = NP // PCH
    for lb2 in range(2):
        b2 = 2 * ci + lb2
        for r in range(-(-CO // 16)):
            c = sid + 16 * r

            @pl.when(c < CO)
            def _task(c=c, b2=b2, lb2=lb2):
                is_xyz = c < 3

                @pl.when(is_xyz)
                def _():
                    pltpu.sync_copy(
                        xyz_f.at[pl.ds((b2 * 3 + c) * N, N)], row_v)
                    pltpu.sync_copy(
                        cen_f.at[pl.ds((b2 * 3 + c) * NP, NP)], cenrow_v)

                @pl.when(jnp.logical_not(is_xyz))
                def _():
                    pltpu.sync_copy(
                        feats_f.at[pl.ds((b2 * C + c - 3) * N, N)], row_v)

                def idx_copy(i):
                    return pltpu.make_async_copy(
                        idx_sh.at[lb2, pl.ds(i * PCH * S, PCH * S)],
                        idxi_v.at[i % 2], semi[i % 2])

                idx_copy(0).start()
                out_descs = [None, None]
                for i, p0 in enumerate(range(0, NP, PCH)):
                    buf = i % 2
                    idx_copy(i).wait()
                    if i + 1 < n_chunks:
                        idx_copy(i + 1).start()
                    if out_descs[buf] is not None:
                        out_descs[buf].wait()

                    @plsc.parallel_loop(0, PCH, unroll=4)
                    def _gather(q):
                        cv = plsc.load_gather(
                            cenrow_v, [jnp.full((L,), p0 + q, jnp.int32)])
                        for h in range(S // L):
                            iv = idxi_v[buf, pl.ds(q * S + h * L, L)]
                            vals = plsc.load_gather(row_v, [iv])
                            vals = jnp.where(is_xyz, vals - cv, vals)
                            ob_v[buf, pl.ds(q * S + h * L, L)] = vals

                    dst = out.at[pl.ds(((b2 * CO + c) * NP + p0) * S,
                                       PCH * S)]
                    out_descs[buf] = pltpu.make_async_copy(
                        ob_v.at[buf], dst, semo[buf])
                    out_descs[buf].start()
                for d in out_descs:
                    if d is not None:
                        d.wait()


@functools.cache
def _qag():
    return pl.kernel(
        _qag_body,
        out_type=jax.ShapeDtypeStruct((B * CO * NP * S,), jnp.float32),
        mesh=plsc.VectorSubcoreMesh(core_axis_name="c", subcore_axis_name="s",
                                    num_cores=2, num_subcores=16),
        compiler_params=pltpu.CompilerParams(needs_layout_passes=False),
        scratch_types=[
            pltpu.VMEM((3 * N,), jnp.float32),
            pltpu.VMEM((3 * NP,), jnp.float32),
            pltpu.VMEM((2 * BUFSZ,), jnp.int32),
            pltpu.VMEM((FLUSH * S,), jnp.int32),
            pltpu.VMEM((N,), jnp.float32),
            pltpu.VMEM((NP,), jnp.float32),
            pltpu.VMEM((2, PCH * S), jnp.int32),
            pltpu.VMEM((2, PCH * S), jnp.float32),
            pltpu.SemaphoreType.DMA,
            pltpu.SemaphoreType.DMA,
            pltpu.SemaphoreType.DMA,
            pltpu.SemaphoreType.DMA,
            pltpu.VMEM_SHARED((2, NP * S), jnp.int32),
        ],
    )


def kernel(xyz, new_xyz, features):
    xyz_f = jnp.transpose(xyz, (0, 2, 1)).reshape(-1)
    cen_f = jnp.transpose(new_xyz, (0, 2, 1)).reshape(-1)
    feats_f = features.reshape(-1)
    out = _qag()(xyz_f, cen_f, feats_f)
    return out.reshape(B, CO, NP, S)


# speedup vs baseline: 41.3368x; 1.2040x over previous
import functools

import jax
import jax.numpy as jnp
import numpy as np
from jax import lax
from jax.experimental import pallas as pl
from jax.experimental.pallas import tpu as pltpu
from jax.experimental.pallas import tpu_sc as plsc

B, N, NP, C = 4, 8192, 2048, 64
S = 32
CO = C + 3
L = 16
TPB = 8
CPT = NP // TPB
GPM = 8
MACROS = N // (GPM * L)
BUFSZ = 160
FLUSH = 64
PCH = 128
RADIUS2 = np.float32(0.2 * 0.2)


def _qag_body(xyz_f, cen_f, feats_f, out,
              xyz_v, cen_v, buf_v, idxc_v, row_v, cenrow_v, idxi_v, ob_v,
              semi0, semi1, semo0, semo1, idx_sh):
    ci = lax.axis_index("c")
    sid = lax.axis_index("s")
    zeros16 = jnp.zeros((L,), jnp.int32)
    iota16 = lax.iota(jnp.int32, L)

    lb = sid // TPB
    b = 2 * ci + lb
    c0 = (sid % TPB) * CPT
    pltpu.sync_copy(xyz_f.at[pl.ds(b * 3 * N, 3 * N)], xyz_v)
    pltpu.sync_copy(cen_f.at[pl.ds(b * 3 * NP, 3 * NP)], cen_v)

    def pair_body(blk, ip, carry0):
        gA = blk * FLUSH + 2 * ip
        colA = jnp.full((L,), c0 + gA, jnp.int32)
        colB = colA + 1
        cxa = plsc.load_gather(cen_v, [colA])
        cya = plsc.load_gather(cen_v, [colA + NP])
        cza = plsc.load_gather(cen_v, [colA + 2 * NP])
        cxb = plsc.load_gather(cen_v, [colB])
        cyb = plsc.load_gather(cen_v, [colB + NP])
        czb = plsc.load_gather(cen_v, [colB + 2 * NP])
        buf_v[pl.ds(0, L)] = zeros16
        buf_v[pl.ds(BUFSZ, L)] = zeros16

        def cond(st):
            j, need, _, _ = st
            return jnp.logical_and(j < MACROS, need)

        def body(st):
            j, _, cntA, cntB = st
            base = j * (GPM * L)
            for k in range(GPM):
                off = base + k * L
                xv = xyz_v[pl.ds(off, L)]
                yv = xyz_v[pl.ds(N + off, L)]
                zv = xyz_v[pl.ds(2 * N + off, L)]
                iv = iota16 + off
                for cx, cy, cz, bb, which in (
                        (cxa, cya, cza, 0, 0), (cxb, cyb, czb, BUFSZ, 1)):
                    dx = xv - cx
                    dy = yv - cy
                    dz = zv - cz
                    d2 = dx * dx + dy * dy + dz * dz
                    m = d2 < RADIUS2
                    mi = m.astype(jnp.int32)
                    cnt = cntB if which else cntA
                    slot = jnp.minimum(cnt + plsc.cumsum(mi) - 1, BUFSZ - 1)
                    plsc.store_scatter(buf_v, [slot + bb], iv, mask=m)
                    cnt = cnt + plsc.all_reduce_population_count(m)
                    if which:
                        cntB = cnt
                    else:
                        cntA = cnt
            mn = jnp.minimum(cntA, cntB)
            need = mn[0] < S
            return (j + 1, need, cntA, cntB)

        init = (jnp.int32(0), jnp.bool_(True), zeros16, zeros16)
        _j, _need, cntA, cntB = lax.while_loop(cond, body, init)

        for bb, cnt, row in ((0, cntA, 2 * ip), (BUFSZ, cntB, 2 * ip + 1)):
            firstv = plsc.load_gather(buf_v, [zeros16 + bb])
            for h in range(S // L):
                p = iota16 + h * L
                got = plsc.load_gather(buf_v, [p + bb])
                idxc_v[pl.ds(row * S + h * L, L)] = jnp.where(
                    p < cnt, got, firstv)
        return carry0

    for blk in range(CPT // FLUSH):
        lax.fori_loop(0, FLUSH // 2,
                      functools.partial(pair_body, blk), 0)
        pltpu.sync_copy(
            idxc_v,
            idx_sh.at[pl.ds(lb * NP * S + (c0 + blk * FLUSH) * S, FLUSH * S)])
    plsc.subcore_barrier()

    semi = (semi0, semi1)
    semo = (semo0, semo1)
    n_chunks = NP // PCH
    for lb2 in range(2):
        b2 = 2 * ci + lb2
        for r in range(-(-CO // 16)):
            c = sid + 16 * r

            @pl.when(c < CO)
            def _task(c=c, b2=b2, lb2=lb2):
                is_xyz = c < 3

                @pl.when(is_xyz)
                def _():
                    pltpu.sync_copy(
                        xyz_f.at[pl.ds((b2 * 3 + c) * N, N)], row_v)
                    pltpu.sync_copy(
                        cen_f.at[pl.ds((b2 * 3 + c) * NP, NP)], cenrow_v)

                @pl.when(jnp.logical_not(is_xyz))
                def _():
                    pltpu.sync_copy(feats_f.at[b2, c - 3], row_v)

                def idx_copy(i, buf):
                    return pltpu.make_async_copy(
                        idx_sh.at[pl.ds(lb2 * NP * S + i * PCH * S, PCH * S)],
                        idxi_v.at[pl.ds(buf * PCH * S, PCH * S)],
                        semi[buf])

                def out_copy(i, buf):
                    return pltpu.make_async_copy(
                        ob_v.at[buf],
                        out.at[b2, c, pl.ds(i * PCH, PCH), :],
                        semo[buf])

                idx_copy(0, 0).start()

                def chunk2(i2, carry):
                    for buf in range(2):
                        i = i2 * 2 + buf
                        idx_copy(i, buf).wait()

                        @pl.when(i + 1 < n_chunks)
                        def _(i=i, buf=buf):
                            idx_copy(i + 1, 1 - buf).start()

                        @pl.when(i >= 2)
                        def _(i=i, buf=buf):
                            out_copy(i - 2, buf).wait()

                        @plsc.parallel_loop(0, PCH, unroll=4)
                        def _gather(q, i=i, buf=buf):
                            cv = plsc.load_gather(
                                cenrow_v,
                                [jnp.full((L,), i * PCH + q, jnp.int32)])
                            for h in range(S // L):
                                iv = idxi_v[pl.ds(
                                    buf * PCH * S + q * S + h * L, L)]
                                vals = plsc.load_gather(row_v, [iv])
                                vals = jnp.where(is_xyz, vals - cv, vals)
                                ob_v[buf, q, pl.ds(h * L, L)] = vals

                        out_copy(i, buf).start()
                    return carry

                lax.fori_loop(0, n_chunks // 2, chunk2, 0)
                out_copy(n_chunks - 2, 0).wait()
                out_copy(n_chunks - 1, 1).wait()


@functools.cache
def _qag():
    return pl.kernel(
        _qag_body,
        out_type=jax.ShapeDtypeStruct((B, CO, NP, S), jnp.float32),
        mesh=plsc.VectorSubcoreMesh(core_axis_name="c", subcore_axis_name="s",
                                    num_cores=2, num_subcores=16),
        compiler_params=pltpu.CompilerParams(needs_layout_passes=False,
                                             use_tc_tiling_on_sc=True),
        scratch_types=[
            pltpu.VMEM((3 * N,), jnp.float32),
            pltpu.VMEM((3 * NP,), jnp.float32),
            pltpu.VMEM((2 * BUFSZ,), jnp.int32),
            pltpu.VMEM((FLUSH * S,), jnp.int32),
            pltpu.VMEM((N,), jnp.float32),
            pltpu.VMEM((NP,), jnp.float32),
            pltpu.VMEM((2 * PCH * S,), jnp.int32),
            pltpu.VMEM((2, PCH, S), jnp.float32),
            pltpu.SemaphoreType.DMA,
            pltpu.SemaphoreType.DMA,
            pltpu.SemaphoreType.DMA,
            pltpu.SemaphoreType.DMA,
            pltpu.VMEM_SHARED((2 * NP * S,), jnp.int32),
        ],
    )


def kernel(xyz, new_xyz, features):
    xyz_f = jnp.transpose(xyz, (0, 2, 1)).reshape(-1)
    cen_f = jnp.transpose(new_xyz, (0, 2, 1)).reshape(-1)
    return _qag()(xyz_f, cen_f, features)
